# baseline (speedup 1.0000x reference)
import jax
import jax.numpy as jnp
from jax import lax
from jax.experimental import pallas as pl
from jax.experimental.pallas import tpu as pltpu
from jax.experimental.pallas import tpu_sc as plsc

MARGIN_ = 1.0
N_ROWS = 128
N_COLS = 32768
L = 16
NC = 2
NS = 16
NW = NC * NS
ROWS_PER_W = N_ROWS // NW
NVEC = N_COLS // L
NEG_INF = float("-inf")
BIG_IDX = 2**30


GV = 32
NG = NVEC // GV
GE = GV * L
N_CHUNKS = 8
CH = N_COLS // N_CHUNKS


def _lane_reduce(vec, red_ref, op):
    for s in (8, 4, 2, 1):
        red_ref[pl.ds(0, L)] = vec
        red_ref[pl.ds(L, L)] = vec
        vec = op(vec, red_ref[pl.ds(s, L)])
    return vec[0]


def _first_group_with(bmax_ref, red_i, target):
    tv = jnp.broadcast_to(target, (L,))

    def body(g, fg):
        hit8 = []
        for u in range(8):
            eq = bmax_ref[pl.ds((g * 8 + u) * L, L)] == tv
            hit8.append(jnp.where(eq, jnp.int32(g * 8 + u), jnp.int32(BIG_IDX)))
        h = jnp.minimum(
            jnp.minimum(jnp.minimum(hit8[0], hit8[1]),
                        jnp.minimum(hit8[2], hit8[3])),
            jnp.minimum(jnp.minimum(hit8[4], hit8[5]),
                        jnp.minimum(hit8[6], hit8[7])),
        )
        return jnp.minimum(fg, h)

    fg = lax.fori_loop(0, NG // 8, body, jnp.full((L,), BIG_IDX, jnp.int32))
    return _lane_reduce(fg, red_i, jnp.minimum)


def _row_top2(row_ref, bmax_ref, red_f, red_i, chunk_cps=None):
    iota = lax.iota(jnp.int32, L)
    ninf = jnp.float32(NEG_INF)

    def gbody(g, m):
        base = g * GE
        acc = [row_ref[pl.ds(base + a * L, L)] for a in range(4)]
        for j in range(4, GV):
            acc[j % 4] = jnp.maximum(acc[j % 4], row_ref[pl.ds(base + j * L, L)])
        gmax = jnp.maximum(jnp.maximum(acc[0], acc[1]),
                           jnp.maximum(acc[2], acc[3]))
        bmax_ref[pl.ds(g * L, L)] = gmax
        return jnp.maximum(m, gmax)

    m = jnp.full((L,), ninf, jnp.float32)
    if chunk_cps is None:
        m = lax.fori_loop(0, NG, gbody, m)
    else:
        npc = NG // len(chunk_cps)
        for c, cp in enumerate(chunk_cps):
            cp.wait()
            m = lax.fori_loop(c * npc, (c + 1) * npc, gbody, m)
    m1 = _lane_reduce(m, red_f, jnp.maximum)
    m1v = jnp.broadcast_to(m1, (L,))

    g1 = _first_group_with(bmax_ref, red_i, m1)
    base1 = g1 * GE

    def ipbody(j8, ip):
        for u in range(8):
            off = base1 + (j8 * 8 + u) * L
            x = row_ref[pl.ds(off, L)]
            ic = iota + off
            ip = jnp.minimum(ip, jnp.where(x == m1v, ic, jnp.int32(BIG_IDX)))
        return ip

    ip = lax.fori_loop(0, GV // 8, ipbody,
                       jnp.full((L,), BIG_IDX, jnp.int32))
    pos = _lane_reduce(ip, red_i, jnp.minimum)

    def gmbody(j8, gm):
        for u in range(8):
            off = base1 + (j8 * 8 + u) * L
            x = row_ref[pl.ds(off, L)]
            ic = iota + off
            gm = jnp.maximum(gm, jnp.where(ic == pos, ninf, x))
        return gm

    gm = lax.fori_loop(0, GV // 8, gmbody, jnp.full((L,), ninf, jnp.float32))
    bmax_ref[pl.ds(g1 * L, L)] = gm

    def mbody(g, mm):
        b = [bmax_ref[pl.ds((g * 8 + u) * L, L)] for u in range(8)]
        t = jnp.maximum(jnp.maximum(jnp.maximum(b[0], b[1]),
                                    jnp.maximum(b[2], b[3])),
                        jnp.maximum(jnp.maximum(b[4], b[5]),
                                    jnp.maximum(b[6], b[7])))
        return jnp.maximum(mm, t)

    mm = lax.fori_loop(0, NG // 8, mbody, jnp.full((L,), ninf, jnp.float32))
    m2 = _lane_reduce(mm, red_f, jnp.maximum)
    m2v = jnp.broadcast_to(m2, (L,))

    g2 = _first_group_with(bmax_ref, red_i, m2)
    base2 = g2 * GE

    def iqbody(j8, iq):
        for u in range(8):
            off = base2 + (j8 * 8 + u) * L
            x = row_ref[pl.ds(off, L)]
            ic = iota + off
            xm = jnp.where(ic == pos, ninf, x)
            iq = jnp.minimum(iq, jnp.where(xm == m2v, ic, jnp.int32(BIG_IDX)))
        return iq

    iq = lax.fori_loop(0, GV // 8, iqbody,
                       jnp.full((L,), BIG_IDX, jnp.int32))
    neg = _lane_reduce(iq, red_i, jnp.minimum)
    return pos, neg


def _lane_pick(vec, lane):
    score = jnp.float32(0.0)
    for l in range(L):
        score = jnp.where(lane == l, vec[l], score)
    return score


def _sc_body(merged_hbm, single_hbm, out_hbm, rowbuf, gbufs, lossbuf, bmax,
             red_f, red_i, sem0, sem1, gsem):
    cid = lax.axis_index("c")
    sid = lax.axis_index("s")
    wid = sid * NC + cid
    iota = lax.iota(jnp.int32, L)
    sems = [sem0, sem1]

    rows = [wid * ROWS_PER_W + k for k in range(ROWS_PER_W)]

    QC = N_COLS // 4
    cps = [None, None]
    cps[0] = [
        pltpu.async_copy(
            single_hbm.at[rows[0], pl.ds(c * QC, QC)],
            rowbuf.at[0, pl.ds(c * QC, QC)],
            sems[0],
        )
        for c in range(4)
    ]
    gcps = []
    lanes = []
    for k in range(ROWS_PER_W):
        slot = k % 2
        if k + 1 < ROWS_PER_W:
            nslot = 1 - slot
            cps[nslot] = pltpu.async_copy(
                single_hbm.at[rows[k + 1]], rowbuf.at[nslot], sems[nslot]
            )
        if k == 0:
            pos, neg = _row_top2(rowbuf.at[slot], bmax, red_f, red_i,
                                 chunk_cps=cps[0])
        else:
            cps[slot].wait()
            pos, neg = _row_top2(rowbuf.at[slot], bmax, red_f, red_i)
        c0p = (pos // L) * L
        c0n = (neg // L) * L
        gcps.append(pltpu.async_copy(
            merged_hbm.at[rows[k], pl.ds(c0p, L)], gbufs.at[2 * k], gsem))
        gcps.append(pltpu.async_copy(
            merged_hbm.at[rows[k], pl.ds(c0n, L)], gbufs.at[2 * k + 1], gsem))
        lanes.append((pos - c0p, neg - c0n))

    for c in gcps:
        c.wait()

    loss_acc = jnp.float32(0.0)
    for k in range(ROWS_PER_W):
        pos_score = _lane_pick(gbufs[2 * k], lanes[k][0])
        neg_score = _lane_pick(gbufs[2 * k + 1], lanes[k][1])
        loss_acc = loss_acc + jnp.maximum(
            jnp.float32(MARGIN_) - (pos_score - neg_score), jnp.float32(0.0)
        )

    lossbuf[...] = jnp.where(
        iota == 0, loss_acc * jnp.float32(1.0 / N_ROWS), jnp.float32(0.0)
    )
    pltpu.sync_copy(lossbuf, out_hbm.at[wid])


@jax.jit
def _distill_loss_sc(merged, single):
    mesh = plsc.VectorSubcoreMesh(core_axis_name="c", subcore_axis_name="s")
    out = pl.kernel(
        _sc_body,
        out_type=jax.ShapeDtypeStruct((NW, L), jnp.float32),
        mesh=mesh,
        scratch_types=[
            pltpu.VMEM((2, N_COLS), jnp.float32),
            pltpu.VMEM((2 * ROWS_PER_W, L), jnp.float32),
            pltpu.VMEM((L,), jnp.float32),
            pltpu.VMEM((NG * L,), jnp.float32),
            pltpu.VMEM((2 * L,), jnp.float32),
            pltpu.VMEM((2 * L,), jnp.int32),
            pltpu.SemaphoreType.DMA,
            pltpu.SemaphoreType.DMA,
            pltpu.SemaphoreType.DMA,
        ],
    )(merged, single)
    return jnp.sum(out)


def kernel(merged_model_output, single_model_output):
    return _distill_loss_sc(merged_model_output, single_model_output)


# speedup vs baseline: 1.0282x; 1.0282x over previous
import jax
import jax.numpy as jnp
from jax import lax
from jax.experimental import pallas as pl
from jax.experimental.pallas import tpu as pltpu
from jax.experimental.pallas import tpu_sc as plsc

MARGIN_ = 1.0
N_ROWS = 128
N_COLS = 32768
L = 16
NC = 2
NS = 16
NW = NC * NS
ROWS_PER_W = N_ROWS // NW
NVEC = N_COLS // L
NEG_INF = float("-inf")
BIG_IDX = 2**30


GV = 32
NG = NVEC // GV
GE = GV * L
N_CHUNKS = 8
CH = N_COLS // N_CHUNKS


def _lane_reduce(vec, red_ref, op):
    for s in (8, 4, 2, 1):
        red_ref[pl.ds(0, L)] = vec
        red_ref[pl.ds(L, L)] = vec
        vec = op(vec, red_ref[pl.ds(s, L)])
    return vec[0]


def _first_group_with(bmax_ref, red_i, target):
    tv = jnp.broadcast_to(target, (L,))

    def body(g, fg):
        hit8 = []
        for u in range(8):
            eq = bmax_ref[pl.ds((g * 8 + u) * L, L)] == tv
            hit8.append(jnp.where(eq, jnp.int32(g * 8 + u), jnp.int32(BIG_IDX)))
        h = jnp.minimum(
            jnp.minimum(jnp.minimum(hit8[0], hit8[1]),
                        jnp.minimum(hit8[2], hit8[3])),
            jnp.minimum(jnp.minimum(hit8[4], hit8[5]),
                        jnp.minimum(hit8[6], hit8[7])),
        )
        return jnp.minimum(fg, h)

    fg = lax.fori_loop(0, NG // 8, body, jnp.full((L,), BIG_IDX, jnp.int32))
    return _lane_reduce(fg, red_i, jnp.minimum)


def _row_top2(row_ref, bmax_ref, red_f, red_i):
    iota = lax.iota(jnp.int32, L)
    ninf = jnp.float32(NEG_INF)

    def gbody(g, m):
        base = g * GE
        acc = [row_ref[pl.ds(base + a * L, L)] for a in range(4)]
        for j in range(4, GV):
            acc[j % 4] = jnp.maximum(acc[j % 4], row_ref[pl.ds(base + j * L, L)])
        gmax = jnp.maximum(jnp.maximum(acc[0], acc[1]),
                           jnp.maximum(acc[2], acc[3]))
        bmax_ref[pl.ds(g * L, L)] = gmax
        return jnp.maximum(m, gmax)

    m = lax.fori_loop(0, NG, gbody, jnp.full((L,), ninf, jnp.float32))
    m1 = _lane_reduce(m, red_f, jnp.maximum)
    m1v = jnp.broadcast_to(m1, (L,))

    g1 = _first_group_with(bmax_ref, red_i, m1)
    base1 = g1 * GE

    def ipbody(j8, ip):
        for u in range(8):
            off = base1 + (j8 * 8 + u) * L
            x = row_ref[pl.ds(off, L)]
            ic = iota + off
            ip = jnp.minimum(ip, jnp.where(x == m1v, ic, jnp.int32(BIG_IDX)))
        return ip

    ip = lax.fori_loop(0, GV // 8, ipbody,
                       jnp.full((L,), BIG_IDX, jnp.int32))
    pos = _lane_reduce(ip, red_i, jnp.minimum)

    def gmbody(j8, gm):
        for u in range(8):
            off = base1 + (j8 * 8 + u) * L
            x = row_ref[pl.ds(off, L)]
            ic = iota + off
            gm = jnp.maximum(gm, jnp.where(ic == pos, ninf, x))
        return gm

    gm = lax.fori_loop(0, GV // 8, gmbody, jnp.full((L,), ninf, jnp.float32))
    bmax_ref[pl.ds(g1 * L, L)] = gm

    def mbody(g, mm):
        b = [bmax_ref[pl.ds((g * 8 + u) * L, L)] for u in range(8)]
        t = jnp.maximum(jnp.maximum(jnp.maximum(b[0], b[1]),
                                    jnp.maximum(b[2], b[3])),
                        jnp.maximum(jnp.maximum(b[4], b[5]),
                                    jnp.maximum(b[6], b[7])))
        return jnp.maximum(mm, t)

    mm = lax.fori_loop(0, NG // 8, mbody, jnp.full((L,), ninf, jnp.float32))
    m2 = _lane_reduce(mm, red_f, jnp.maximum)
    m2v = jnp.broadcast_to(m2, (L,))

    g2 = _first_group_with(bmax_ref, red_i, m2)
    base2 = g2 * GE

    def iqbody(j8, iq):
        for u in range(8):
            off = base2 + (j8 * 8 + u) * L
            x = row_ref[pl.ds(off, L)]
            ic = iota + off
            xm = jnp.where(ic == pos, ninf, x)
            iq = jnp.minimum(iq, jnp.where(xm == m2v, ic, jnp.int32(BIG_IDX)))
        return iq

    iq = lax.fori_loop(0, GV // 8, iqbody,
                       jnp.full((L,), BIG_IDX, jnp.int32))
    neg = _lane_reduce(iq, red_i, jnp.minimum)
    return pos, neg


def _lane_pick(vec, lane):
    score = jnp.float32(0.0)
    for l in range(L):
        score = jnp.where(lane == l, vec[l], score)
    return score


def _sc_body(merged_hbm, single_hbm, out_hbm, rowbuf, gbufs, lossbuf, bmax,
             red_f, red_i, sem0, sem1, gsem):
    cid = lax.axis_index("c")
    sid = lax.axis_index("s")
    wid = sid * NC + cid
    iota = lax.iota(jnp.int32, L)
    sems = [sem0, sem1]

    rows = [wid * ROWS_PER_W + k for k in range(ROWS_PER_W)]

    cps = [None, None]
    cps[0] = pltpu.async_copy(single_hbm.at[rows[0]], rowbuf.at[0], sems[0])
    gcps = []
    lanes = []
    for k in range(ROWS_PER_W):
        slot = k % 2
        if k + 1 < ROWS_PER_W:
            nslot = 1 - slot
            cps[nslot] = pltpu.async_copy(
                single_hbm.at[rows[k + 1]], rowbuf.at[nslot], sems[nslot]
            )
        cps[slot].wait()
        pos, neg = _row_top2(rowbuf.at[slot], bmax, red_f, red_i)
        c0p = (pos // L) * L
        c0n = (neg // L) * L
        gcps.append(pltpu.async_copy(
            merged_hbm.at[rows[k], pl.ds(c0p, L)], gbufs.at[2 * k], gsem))
        gcps.append(pltpu.async_copy(
            merged_hbm.at[rows[k], pl.ds(c0n, L)], gbufs.at[2 * k + 1], gsem))
        lanes.append((pos - c0p, neg - c0n))

    for c in gcps:
        c.wait()

    loss_acc = jnp.float32(0.0)
    for k in range(ROWS_PER_W):
        pos_score = _lane_pick(gbufs[2 * k], lanes[k][0])
        neg_score = _lane_pick(gbufs[2 * k + 1], lanes[k][1])
        loss_acc = loss_acc + jnp.maximum(
            jnp.float32(MARGIN_) - (pos_score - neg_score), jnp.float32(0.0)
        )

    lossbuf[...] = jnp.where(
        iota == 0, loss_acc * jnp.float32(1.0 / N_ROWS), jnp.float32(0.0)
    )
    pltpu.sync_copy(lossbuf, out_hbm.at[wid])


@jax.jit
def _distill_loss_sc(merged, single):
    mesh = plsc.VectorSubcoreMesh(core_axis_name="c", subcore_axis_name="s")
    out = pl.kernel(
        _sc_body,
        out_type=jax.ShapeDtypeStruct((NW, L), jnp.float32),
        mesh=mesh,
        scratch_types=[
            pltpu.VMEM((2, N_COLS), jnp.float32),
            pltpu.VMEM((2 * ROWS_PER_W, L), jnp.float32),
            pltpu.VMEM((L,), jnp.float32),
            pltpu.VMEM((NG * L,), jnp.float32),
            pltpu.VMEM((2 * L,), jnp.float32),
            pltpu.VMEM((2 * L,), jnp.int32),
            pltpu.SemaphoreType.DMA,
            pltpu.SemaphoreType.DMA,
            pltpu.SemaphoreType.DMA,
        ],
    )(merged, single)
    return jnp.sum(out)


def kernel(merged_model_output, single_model_output):
    return _distill_loss_sc(merged_model_output, single_model_output)
